# baseline (speedup 1.0000x reference)
import functools

import jax
import jax.numpy as jnp
from jax import lax
from jax.experimental import pallas as pl
from jax.experimental.pallas import tpu as pltpu
from jax.experimental.pallas import tpu_sc as plsc

_INTER = 512
_N = 128
_E = 8
_K = 2
_LANES = 16


def _route_body(idx_hbm, wts_hbm, w_hbm, idx_v, wts_v, w_v):
    cid = lax.axis_index("c")
    sid = lax.axis_index("s")

    @pl.when((cid == 0) & (sid == 0))
    def _():
        pltpu.sync_copy(idx_hbm, idx_v)
        pltpu.sync_copy(wts_hbm, wts_v)
        for i in range(_N * _E // _LANES):
            w_v[pl.ds(i * _LANES, _LANES)] = jnp.zeros((_LANES,), jnp.float32)
        lane = lax.iota(jnp.int32, _LANES)
        even = (lane & 1) == 0
        for j in range(_N * _K // _LANES):
            s0 = j * _LANES
            idx16 = idx_v[pl.ds(s0, _LANES)]
            w16 = wts_v[pl.ds(s0, _LANES)]
            tok = lax.shift_right_logical(s0 + lane, 1)
            flat = tok * _E + idx16
            plsc.addupdate_scatter(w_v, [flat], w16, mask=even)
            plsc.addupdate_scatter(w_v, [flat], w16, mask=~even)
        pltpu.sync_copy(w_v, w_hbm)


_route = functools.partial(
    pl.kernel,
    _route_body,
    out_type=jax.ShapeDtypeStruct((_N * _E,), jnp.float32),
    mesh=plsc.VectorSubcoreMesh(core_axis_name="c", subcore_axis_name="s"),
    compiler_params=pltpu.CompilerParams(needs_layout_passes=False),
    scratch_types=[
        pltpu.VMEM((_N * _K,), jnp.int32),
        pltpu.VMEM((_N * _K,), jnp.float32),
        pltpu.VMEM((_N * _E,), jnp.float32),
    ],
)()


def _moe_body(x_ref, gu_ref, dn_ref, w_ref, out_ref):
    e = pl.program_id(0)
    x = x_ref[...]
    proj = jax.lax.dot_general(
        x, gu_ref[0], (((1,), (1,)), ((), ())),
        preferred_element_type=jnp.float32)
    gate = proj[:, :_INTER]
    up = proj[:, _INTER:]
    h = gate * jax.nn.sigmoid(gate) * up
    out_e = jax.lax.dot_general(
        h, dn_ref[0], (((1,), (1,)), ((), ())),
        preferred_element_type=jnp.float32)
    lane = lax.broadcasted_iota(jnp.int32, (_N, _E), 1)
    w = jnp.sum(w_ref[...] * (lane == e).astype(jnp.float32),
                axis=1, keepdims=True)
    contrib = out_e * w

    @pl.when(e == 0)
    def _():
        out_ref[...] = contrib

    @pl.when(e != 0)
    def _():
        out_ref[...] += contrib


@jax.jit
def kernel(hidden_states, top_k_index, top_k_weights, gate_up_proj, down_proj):
    n, h = hidden_states.shape
    e = gate_up_proj.shape[0]
    i2 = gate_up_proj.shape[1]
    i = down_proj.shape[2]
    idx_flat = top_k_index.astype(jnp.int32).reshape(-1)
    wts_flat = top_k_weights.reshape(-1)
    w_dense = _route(idx_flat, wts_flat).reshape(n, e)
    out = pl.pallas_call(
        _moe_body,
        grid=(e,),
        in_specs=[
            pl.BlockSpec((n, h), lambda ei: (0, 0)),
            pl.BlockSpec((1, i2, h), lambda ei: (ei, 0, 0)),
            pl.BlockSpec((1, h, i), lambda ei: (ei, 0, 0)),
            pl.BlockSpec((n, e), lambda ei: (0, 0)),
        ],
        out_specs=pl.BlockSpec((n, h), lambda ei: (0, 0)),
        out_shape=jax.ShapeDtypeStruct((n, h), jnp.float32),
    )(hidden_states, gate_up_proj, down_proj, w_dense)
    return out.astype(hidden_states.dtype)


# speedup vs baseline: 1.7864x; 1.7864x over previous
import functools

import jax
import jax.numpy as jnp
from jax.experimental import pallas as pl

_INTER = 512


def _moe_body(x_ref, gu_ref, dn_ref, idx_ref, wts_ref, out_ref):
    e = pl.program_id(0)
    x = x_ref[...]
    proj = jax.lax.dot_general(
        x, gu_ref[0], (((1,), (1,)), ((), ())),
        preferred_element_type=jnp.float32)
    gate = proj[:, :_INTER]
    up = proj[:, _INTER:]
    h = gate * jax.nn.sigmoid(gate) * up
    out_e = jax.lax.dot_general(
        h, dn_ref[0], (((1,), (1,)), ((), ())),
        preferred_element_type=jnp.float32)
    sel = (idx_ref[...] == e).astype(jnp.float32)
    w = jnp.sum(wts_ref[...] * sel, axis=1, keepdims=True)
    contrib = out_e * w

    @pl.when(e == 0)
    def _():
        out_ref[...] = contrib

    @pl.when(e != 0)
    def _():
        out_ref[...] += contrib


@jax.jit
def kernel(hidden_states, top_k_index, top_k_weights, gate_up_proj, down_proj):
    n, h = hidden_states.shape
    e = gate_up_proj.shape[0]
    i2 = gate_up_proj.shape[1]
    i = down_proj.shape[2]
    out = pl.pallas_call(
        _moe_body,
        grid=(e,),
        in_specs=[
            pl.BlockSpec((n, h), lambda ei: (0, 0)),
            pl.BlockSpec((1, i2, h), lambda ei: (ei, 0, 0)),
            pl.BlockSpec((1, h, i), lambda ei: (ei, 0, 0)),
            pl.BlockSpec(top_k_index.shape, lambda ei: (0, 0)),
            pl.BlockSpec(top_k_weights.shape, lambda ei: (0, 0)),
        ],
        out_specs=pl.BlockSpec((n, h), lambda ei: (0, 0)),
        out_shape=jax.ShapeDtypeStruct((n, h), jnp.float32),
    )(hidden_states, gate_up_proj, down_proj,
      top_k_index.astype(jnp.int32), top_k_weights)
    return out.astype(hidden_states.dtype)


# speedup vs baseline: 1.9847x; 1.1110x over previous
import functools

import jax
import jax.numpy as jnp
from jax.experimental import pallas as pl
from jax.experimental.pallas import tpu as pltpu

_INTER = 512
_E = 8


def _moe_body(x_ref, gu_hbm, dn_hbm, idx_ref, wts_ref, out_ref,
              gu_buf, dn_buf, gu_sem, dn_sem):
    def gu_copy(e):
        return pltpu.make_async_copy(
            gu_hbm.at[e], gu_buf.at[e % 2], gu_sem.at[e % 2])

    def dn_copy(e):
        return pltpu.make_async_copy(
            dn_hbm.at[e], dn_buf.at[e % 2], dn_sem.at[e % 2])

    gu_copy(0).start()
    dn_copy(0).start()
    x = x_ref[...]
    for e in range(_E):
        if e + 1 < _E:
            gu_copy(e + 1).start()
            dn_copy(e + 1).start()
        gu_copy(e).wait()
        proj = jax.lax.dot_general(
            x, gu_buf[e % 2], (((1,), (1,)), ((), ())),
            preferred_element_type=jnp.float32)
        gate = proj[:, :_INTER]
        up = proj[:, _INTER:]
        h = gate * jax.nn.sigmoid(gate) * up
        dn_copy(e).wait()
        out_e = jax.lax.dot_general(
            h, dn_buf[e % 2], (((1,), (1,)), ((), ())),
            preferred_element_type=jnp.float32)
        sel = (idx_ref[...] == e).astype(jnp.float32)
        w = jnp.sum(wts_ref[...] * sel, axis=1, keepdims=True)
        contrib = out_e * w
        if e == 0:
            out_ref[...] = contrib
        else:
            out_ref[...] += contrib


@jax.jit
def kernel(hidden_states, top_k_index, top_k_weights, gate_up_proj, down_proj):
    n, h = hidden_states.shape
    e = gate_up_proj.shape[0]
    i2 = gate_up_proj.shape[1]
    i = down_proj.shape[2]
    out = pl.pallas_call(
        _moe_body,
        in_specs=[
            pl.BlockSpec(memory_space=pltpu.MemorySpace.VMEM),
            pl.BlockSpec(memory_space=pltpu.MemorySpace.HBM),
            pl.BlockSpec(memory_space=pltpu.MemorySpace.HBM),
            pl.BlockSpec(memory_space=pltpu.MemorySpace.VMEM),
            pl.BlockSpec(memory_space=pltpu.MemorySpace.VMEM),
        ],
        out_specs=pl.BlockSpec(memory_space=pltpu.MemorySpace.VMEM),
        out_shape=jax.ShapeDtypeStruct((n, h), jnp.float32),
        scratch_shapes=[
            pltpu.VMEM((2, i2, h), jnp.float32),
            pltpu.VMEM((2, h, i), jnp.float32),
            pltpu.SemaphoreType.DMA((2,)),
            pltpu.SemaphoreType.DMA((2,)),
        ],
    )(hidden_states, gate_up_proj, down_proj,
      top_k_index.astype(jnp.int32), top_k_weights)
    return out.astype(hidden_states.dtype)
